# Initial kernel scaffold; baseline (speedup 1.0000x reference)
import jax
import jax.numpy as jnp
from jax.experimental import pallas as pl


def kernel(h, pos_enc, norm, prev_h, emb_rel, W_hp, b_hp, W_neighbor, edge_index, edge_type):
    raise NotImplementedError("write your pallas kernel here")


# speedup vs baseline: 6.7310x; 6.7310x over previous
import functools

import jax
import jax.numpy as jnp
from jax import lax
from jax.experimental import pallas as pl
from jax.experimental.pallas import tpu as pltpu
from jax.experimental.pallas import tpu_sc as plsc

NC = 2
NS = 16
NW = NC * NS


def _sc_mesh():
    return plsc.VectorSubcoreMesh(
        core_axis_name="c", subcore_axis_name="s", num_cores=NC, num_subcores=NS
    )


def _make_edge_scatter(NPAD, E, D, B):
    EPW = E // NW
    NB = EPW // B
    RPT = NPAD // NS

    @functools.partial(
        pl.kernel,
        out_type=jax.ShapeDtypeStruct((NC, NPAD, D), jnp.float32),
        mesh=_sc_mesh(),
        scratch_types=[
            pltpu.VMEM((B,), jnp.int32),
            pltpu.VMEM((B,), jnp.int32),
            pltpu.VMEM((B,), jnp.int32),
            pltpu.VMEM((B, D), jnp.float32),
            pltpu.VMEM((B, D), jnp.float32),
            pltpu.VMEM_SHARED((NPAD, D), jnp.float32),
            pltpu.SemaphoreType.DMA,
            pltpu.SemaphoreType.DMA,
        ],
    )
    def kern(z_hbm, rel_hbm, src_hbm, dst_hbm, et_hbm, zrow_hbm, g_out,
             src_v, dst_v, et_v, zr_v, rr_v, g_sh, sem_a, sem_b):
        c = lax.axis_index("c")
        s = lax.axis_index("s")
        wid = c * NS + s
        pltpu.sync_copy(zrow_hbm, g_sh.at[pl.ds(s * RPT, RPT)])
        plsc.subcore_barrier()

        base = wid * EPW

        def body(i, carry):
            off = base + i * B
            pltpu.sync_copy(src_hbm.at[pl.ds(off, B)], src_v)
            pltpu.sync_copy(et_hbm.at[pl.ds(off, B)], et_v)
            pltpu.sync_copy(dst_hbm.at[pl.ds(off, B)], dst_v)
            cp_a = pltpu.async_copy(z_hbm.at[src_v], zr_v, sem_a)
            cp_b = pltpu.async_copy(rel_hbm.at[et_v], rr_v, sem_b)
            cp_a.wait()
            pltpu.sync_copy(zr_v, g_sh.at[dst_v], add=True)
            cp_b.wait()
            pltpu.sync_copy(rr_v, g_sh.at[dst_v], add=True)
            return carry

        lax.fori_loop(0, NB, body, 0)
        plsc.subcore_barrier()
        pltpu.sync_copy(
            g_sh.at[pl.ds(s * RPT, RPT)], g_out.at[c, pl.ds(s * RPT, RPT)]
        )

    return kern


def _z_body(hb, pb, w1, w2, b2, out):
    out[...] = (
        jnp.dot(hb[...], w1[...], preferred_element_type=jnp.float32)
        + jnp.dot(pb[...], w2[...], preferred_element_type=jnp.float32)
        + b2[...]
    )


def _merge_body(g0, g1, nrm, wn, out):
    gg = g0[...] + g1[...]
    out[...] = jnp.dot(gg, wn[...], preferred_element_type=jnp.float32) * nrm[...]


def kernel(h, pos_enc, norm, prev_h, emb_rel, W_hp, b_hp, W_neighbor, edge_index, edge_type):
    N, D = h.shape
    P = pos_enc.shape[1]
    R = emb_rel.shape[0]
    E = edge_type.shape[0]
    B = 80
    NPAD = 10240
    PP = 8

    posp = jnp.concatenate([pos_enc, jnp.zeros((N, PP - P), jnp.float32)], axis=1)
    w1 = W_hp[:D]
    w2 = jnp.concatenate([W_hp[D:], jnp.zeros((PP - P, D), jnp.float32)], axis=0)
    b2 = b_hp.reshape(1, D)
    src = edge_index[0]
    dst = edge_index[1]
    zrow = jnp.zeros((NPAD // NS, D), jnp.float32)

    BN = 1000
    z = pl.pallas_call(
        _z_body,
        grid=(N // BN,),
        in_specs=[
            pl.BlockSpec((BN, D), lambda i: (i, 0)),
            pl.BlockSpec((BN, PP), lambda i: (i, 0)),
            pl.BlockSpec((D, D), lambda i: (0, 0)),
            pl.BlockSpec((PP, D), lambda i: (0, 0)),
            pl.BlockSpec((1, D), lambda i: (0, 0)),
        ],
        out_specs=pl.BlockSpec((BN, D), lambda i: (i, 0)),
        out_shape=jax.ShapeDtypeStruct((N, D), jnp.float32),
    )(h, posp, w1, w2, b2)

    g_parts = _make_edge_scatter(NPAD, E, D, B)(z, emb_rel, src, dst, edge_type, zrow)

    node_repr = pl.pallas_call(
        _merge_body,
        grid=(N // BN,),
        in_specs=[
            pl.BlockSpec((BN, D), lambda i: (i, 0)),
            pl.BlockSpec((BN, D), lambda i: (i, 0)),
            pl.BlockSpec((BN, 1), lambda i: (i, 0)),
            pl.BlockSpec((D, D), lambda i: (0, 0)),
        ],
        out_specs=pl.BlockSpec((BN, D), lambda i: (i, 0)),
        out_shape=jax.ShapeDtypeStruct((N, D), jnp.float32),
    )(g_parts[0], g_parts[1], norm, W_neighbor)
    return node_repr, pos_enc
